# baseline (speedup 1.0000x reference)
import functools

import jax
import jax.numpy as jnp
from jax import lax
from jax.experimental import pallas as pl
from jax.experimental.pallas import tpu as pltpu
from jax.experimental.pallas import tpu_sc as plsc

EMBED = 32
NUM_WORKERS = 32


def _gather_body(batch, hist, table, idx, out,
                 idx_v, rows_v, col_v, gsem, ssem, isem):
    wid = lax.axis_index("s") * 2 + lax.axis_index("c")
    bpw = batch // NUM_WORKERS
    b0 = wid * bpw
    n_streams = bpw // 128
    n_pairs = hist // 2
    lanes = lax.iota(jnp.int32, 16)
    bsub = [lanes + (16 * i) % 128 for i in range(8)]

    def stage_idx(h, p):
        pltpu.async_copy(idx.at[h, pl.ds(b0, bpw)], idx_v.at[p], isem[p])

    def wait_idx(p):
        pltpu.make_async_copy(idx.at[0, pl.ds(b0, bpw)], idx_v.at[p],
                              isem[p]).wait()

    def fire_gathers(p):
        for s in range(n_streams):
            pltpu.async_copy(
                table.at[idx_v.at[p, pl.ds(s * 128, 128)]],
                rows_v.at[p, pl.ds(s * 128, 128)], gsem[p])

    def drain_gathers(p):
        for s in range(n_streams):
            pltpu.make_async_copy(
                table.at[idx_v.at[p, pl.ds(s * 128, 128)]],
                rows_v.at[p, pl.ds(s * 128, 128)], gsem[p]).wait()

    def transpose(p):
        @plsc.parallel_loop(0, EMBED, unroll=2)
        def t_body(k):
            col = lax.rem(lanes + k, jnp.int32(EMBED))
            ti = lax.shift_right_logical(col, 3)
            ibase = (col & 7) << 7
            for i in range(32):
                ri = lanes + i * 16
                v = plsc.load_gather(rows_v.at[p], [ri, col])
                inner = ibase + ((i // 8) * 1024) + bsub[i % 8]
                plsc.store_scatter(col_v.at[p], [ti, inner], v)

    def store_out(h, p):
        pltpu.async_copy(col_v.at[p],
                         out.at[h, pl.ds(0, EMBED // 8),
                                pl.ds((b0 // 128) * 1024, bpw * 8)], ssem[p])

    def wait_store(p):
        pltpu.make_async_copy(col_v.at[p],
                              out.at[0, pl.ds(0, EMBED // 8),
                                     pl.ds((b0 // 128) * 1024, bpw * 8)],
                              ssem[p]).wait()

    for p in (0, 1):
        stage_idx(p, p)
        wait_idx(p)
        fire_gathers(p)

    def body(g, carry):
        h0 = 2 * g
        for p in (0, 1):
            h = h0 + p
            drain_gathers(p)

            @pl.when(g < n_pairs - 1)
            def _():
                stage_idx(h + 2, p)

            @pl.when(g > 0)
            def _():
                wait_store(p)
            transpose(p)
            store_out(h, p)

            @pl.when(g < n_pairs - 1)
            def _():
                wait_idx(p)
                fire_gathers(p)
        return carry

    lax.fori_loop(0, n_pairs, body, 0)
    wait_store(0)
    wait_store(1)


def kernel(x, embedding):
    batch, hist = x.shape
    idx = jnp.transpose(x.astype(jnp.int32))
    bpw = batch // NUM_WORKERS

    gather = pl.kernel(
        functools.partial(_gather_body, batch, hist),
        out_type=jax.ShapeDtypeStruct((hist, EMBED // 8, 8 * batch),
                                      jnp.float32),
        mesh=plsc.VectorSubcoreMesh(core_axis_name="c", subcore_axis_name="s"),
        scratch_types=[
            pltpu.VMEM((2, bpw), jnp.int32),
            pltpu.VMEM((2, bpw, EMBED), jnp.float32),
            pltpu.VMEM((2, EMBED // 8, bpw * 8), jnp.float32),
            [pltpu.SemaphoreType.DMA, pltpu.SemaphoreType.DMA],
            [pltpu.SemaphoreType.DMA, pltpu.SemaphoreType.DMA],
            [pltpu.SemaphoreType.DMA, pltpu.SemaphoreType.DMA],
        ],
        compiler_params=pltpu.CompilerParams(
            use_tc_tiling_on_sc=False, needs_layout_passes=False),
    )
    out = gather(embedding, idx)
    y = out.reshape(hist, EMBED // 8, batch // 128, 8, 128)
    y = jnp.transpose(y, (2, 4, 0, 1, 3))
    return y.reshape(batch, hist, EMBED)


# speedup vs baseline: 1.1926x; 1.1926x over previous
import functools

import jax
import jax.numpy as jnp
from jax import lax
from jax.experimental import pallas as pl
from jax.experimental.pallas import tpu as pltpu
from jax.experimental.pallas import tpu_sc as plsc

EMBED = 32
NUM_WORKERS = 32


def _gather_body(batch, hist, table, idx, out,
                 idx_v, rows_v, col_v, gsem, ssem, isem):
    wid = lax.axis_index("s") * 2 + lax.axis_index("c")
    bpw = batch // NUM_WORKERS
    b0 = wid * bpw
    n_streams = bpw // 128
    n_pairs = hist // 2
    lanes = lax.iota(jnp.int32, 16)
    bsub = [lanes + (16 * i) % 128 for i in range(8)]

    def stage_idx(h, p):
        pltpu.async_copy(idx.at[h, pl.ds(b0, bpw)], idx_v.at[p], isem[p])

    def wait_idx(p):
        pltpu.make_async_copy(idx.at[0, pl.ds(b0, bpw)], idx_v.at[p],
                              isem[p]).wait()

    def fire_gathers(p):
        for s in range(n_streams):
            pltpu.async_copy(
                table.at[idx_v.at[p, pl.ds(s * 128, 128)]],
                rows_v.at[p, pl.ds(s * 128, 128)], gsem[p])

    def drain_gathers(p):
        for s in range(n_streams):
            pltpu.make_async_copy(
                table.at[idx_v.at[p, pl.ds(s * 128, 128)]],
                rows_v.at[p, pl.ds(s * 128, 128)], gsem[p]).wait()

    def transpose(p):
        @plsc.parallel_loop(0, EMBED)
        def t_body(k):
            col = lax.rem(lanes + k, jnp.int32(EMBED))
            ti = lax.shift_right_logical(col, 3)
            ibase = (col & 7) << 7
            for i in range(32):
                ri = lanes + i * 16
                v = plsc.load_gather(rows_v.at[p], [ri, col])
                inner = ibase + ((i // 8) * 1024) + bsub[i % 8]
                plsc.store_scatter(col_v.at[p], [ti, inner], v)

    def store_out(h, p):
        pltpu.async_copy(col_v.at[p],
                         out.at[h, pl.ds(0, EMBED // 8),
                                pl.ds((b0 // 128) * 1024, bpw * 8)], ssem[p])

    def wait_store(p):
        pltpu.make_async_copy(col_v.at[p],
                              out.at[0, pl.ds(0, EMBED // 8),
                                     pl.ds((b0 // 128) * 1024, bpw * 8)],
                              ssem[p]).wait()

    for p in (0, 1):
        stage_idx(p, p)
        wait_idx(p)
        fire_gathers(p)

    def body(g, carry):
        h0 = 2 * g
        for p in (0, 1):
            h = h0 + p
            drain_gathers(p)

            @pl.when(g < n_pairs - 1)
            def _():
                stage_idx(h + 2, p)

            @pl.when(g > 0)
            def _():
                wait_store(p)
            transpose(p)
            store_out(h, p)

            @pl.when(g < n_pairs - 1)
            def _():
                wait_idx(p)
                fire_gathers(p)
        return carry

    lax.fori_loop(0, n_pairs, body, 0)
    wait_store(0)
    wait_store(1)


def kernel(x, embedding):
    batch, hist = x.shape
    idx = jnp.transpose(x.astype(jnp.int32))
    bpw = batch // NUM_WORKERS

    gather = pl.kernel(
        functools.partial(_gather_body, batch, hist),
        out_type=jax.ShapeDtypeStruct((hist, EMBED // 8, 8 * batch),
                                      jnp.float32),
        mesh=plsc.VectorSubcoreMesh(core_axis_name="c", subcore_axis_name="s"),
        scratch_types=[
            pltpu.VMEM((2, bpw), jnp.int32),
            pltpu.VMEM((2, bpw, EMBED), jnp.float32),
            pltpu.VMEM((2, EMBED // 8, bpw * 8), jnp.float32),
            [pltpu.SemaphoreType.DMA, pltpu.SemaphoreType.DMA],
            [pltpu.SemaphoreType.DMA, pltpu.SemaphoreType.DMA],
            [pltpu.SemaphoreType.DMA, pltpu.SemaphoreType.DMA],
        ],
        compiler_params=pltpu.CompilerParams(
            use_tc_tiling_on_sc=False, needs_layout_passes=False),
    )
    out = gather(embedding, idx)
    y = out.reshape(hist, EMBED // 8, batch // 128, 8, 128)
    y = jnp.transpose(y, (2, 4, 0, 1, 3))
    return y.reshape(batch, hist, EMBED)
